# Initial kernel scaffold; baseline (speedup 1.0000x reference)
import jax
import jax.numpy as jnp
from jax.experimental import pallas as pl


def kernel(x, edge_index, W1, b1, W2, b2, W3, b3):
    raise NotImplementedError("write your pallas kernel here")


# speedup vs baseline: 5.0633x; 5.0633x over previous
import functools

import jax
import jax.numpy as jnp
from jax import lax
from jax.experimental import pallas as pl
from jax.experimental.pallas import tpu as pltpu
from jax.experimental.pallas import tpu_sc as plsc

_N = 10000
_D = 128
_E = 320000
_NCLS = 6
_NC = 2
_NS = 16
_NW = _NC * _NS
_EPT = _E // _NW
_CHUNK = 80
_NCHUNK = _EPT // _CHUNK
_RB = 624
_RREM = _N - _RB * _NS
_ZR = 16


def _sc_agg_body(src_ref, dst_ref, x_ref, out_ref, src_v, dst_v, rows_v,
                 zero_v, agg_sh, sem):
    c = lax.axis_index("c")
    s = lax.axis_index("s")
    wid = c * _NS + s

    zvec = jnp.zeros((16,), jnp.float32)
    for i in range(_ZR):
        for j in range(_D // 16):
            zero_v[i, pl.ds(j * 16, 16)] = zvec

    def zero_body(i, carry):
        pltpu.sync_copy(zero_v, agg_sh.at[pl.ds(s * _RB + i * _ZR, _ZR)])
        return carry

    lax.fori_loop(0, _RB // _ZR, zero_body, 0)

    @pl.when(s == _NS - 1)
    def _zero_rem():
        pltpu.sync_copy(zero_v, agg_sh.at[pl.ds(_RB * _NS, _RREM)])

    plsc.subcore_barrier()

    base = wid * _EPT

    def chunk_body(i, carry):
        off = base + i * _CHUNK
        pltpu.sync_copy(src_ref.at[pl.ds(off, _CHUNK)], src_v)
        pltpu.sync_copy(dst_ref.at[pl.ds(off, _CHUNK)], dst_v)
        pltpu.async_copy(x_ref.at[src_v], rows_v, sem).wait()
        pltpu.sync_copy(rows_v, agg_sh.at[dst_v], add=True)
        return carry

    lax.fori_loop(0, _NCHUNK, chunk_body, 0)
    plsc.subcore_barrier()

    pltpu.sync_copy(agg_sh.at[pl.ds(s * _RB, _RB)],
                    out_ref.at[c, pl.ds(s * _RB, _RB)])

    @pl.when(s == _NS - 1)
    def _copy_rem():
        pltpu.sync_copy(agg_sh.at[pl.ds(_RB * _NS, _RREM)],
                        out_ref.at[c, pl.ds(_RB * _NS, _RREM)])


@functools.lru_cache(maxsize=1)
def _sc_agg():
    return pl.kernel(
        _sc_agg_body,
        out_type=jax.ShapeDtypeStruct((_NC, _N, _D), jnp.float32),
        mesh=plsc.VectorSubcoreMesh(core_axis_name="c", subcore_axis_name="s",
                                    num_cores=_NC, num_subcores=_NS),
        scratch_types=[
            pltpu.VMEM((_CHUNK,), jnp.int32),
            pltpu.VMEM((_CHUNK,), jnp.int32),
            pltpu.VMEM((_CHUNK, _D), jnp.float32),
            pltpu.VMEM((_ZR, _D), jnp.float32),
            pltpu.VMEM_SHARED((_N, _D), jnp.float32),
            pltpu.SemaphoreType.DMA,
        ],
    )


def _mlp_body(x_ref, agg_ref, w1_ref, b1_ref, w2_ref, b2_ref, w3_ref, b3_ref,
              out_ref):
    h = agg_ref[0] + agg_ref[1] + x_ref[...]
    h = jnp.dot(h, w1_ref[...], preferred_element_type=jnp.float32)
    h = jnp.maximum(h + b1_ref[...], 0.0)
    h = jnp.dot(h, w2_ref[...], preferred_element_type=jnp.float32)
    h = jnp.maximum(h + b2_ref[...], 0.0)
    colsum = jnp.sum(h, axis=0, keepdims=True)
    out_ref[...] = (jnp.dot(colsum, w3_ref[...],
                            preferred_element_type=jnp.float32) + b3_ref[...])


_mlp = pl.pallas_call(
    _mlp_body,
    out_shape=jax.ShapeDtypeStruct((1, _D), jnp.float32),
)


def kernel(x, edge_index, W1, b1, W2, b2, W3, b3):
    ei = edge_index.astype(jnp.int32)
    agg = _sc_agg()(ei[0], ei[1], x)
    w3t = jnp.zeros((_D, _D), jnp.float32).at[:, :_NCLS].set(W3.T)
    b3p = jnp.zeros((1, _D), jnp.float32).at[0, :_NCLS].set(b3 * _N)
    y = _mlp(x, agg, W1.T, b1.reshape(1, _D), W2.T, b2.reshape(1, _D),
             w3t, b3p)
    return y[0, :_NCLS]


# speedup vs baseline: 9.5995x; 1.8959x over previous
import functools

import jax
import jax.numpy as jnp
from jax import lax
from jax.experimental import pallas as pl
from jax.experimental.pallas import tpu as pltpu
from jax.experimental.pallas import tpu_sc as plsc

_N = 10000
_D = 128
_E = 320000
_NCLS = 6
_NC = 2
_NS = 16
_NW = _NC * _NS
_EPT = _E // _NW
_CHUNK = 100
_NCHUNK = _EPT // _CHUNK
_NH = 2
_HC = _NCHUNK // _NH
_RB = 624
_RREM = _N - _RB * _NS
_ZR = 16


def _sc_agg_body(src_ref, dst_ref, x_ref, out_ref, src_all, dst_all, rows0,
                 rows1, zero_v, agg_sh, sem_i, sem0, sem1):
    c = lax.axis_index("c")
    s = lax.axis_index("s")
    wid = c * _NS + s

    idx_cp0 = pltpu.async_copy(src_ref.at[wid, 0], src_all, sem_i)
    idx_cp1 = pltpu.async_copy(dst_ref.at[wid, 0], dst_all, sem_i)

    zvec = jnp.zeros((16,), jnp.float32)
    for i in range(_ZR):
        for j in range(_D // 16):
            zero_v[i, pl.ds(j * 16, 16)] = zvec

    def zero_body(i, carry):
        pltpu.sync_copy(zero_v, agg_sh.at[pl.ds(s * _RB + i * _ZR, _ZR)])
        return carry

    lax.fori_loop(0, _RB // _ZR, zero_body, 0)

    @pl.when(s == _NS - 1)
    def _zero_rem():
        pltpu.sync_copy(zero_v, agg_sh.at[pl.ds(_RB * _NS, _RREM)])

    idx_cp0.wait()
    idx_cp1.wait()
    plsc.subcore_barrier()

    def chunk_body(j, carry):
        a = 2 * j
        b = a + 1
        pltpu.make_async_copy(x_ref.at[src_all.at[a]], rows0, sem0).wait()
        pltpu.async_copy(x_ref.at[src_all.at[b]], rows1, sem1)
        pltpu.sync_copy(rows0, agg_sh.at[dst_all.at[a]], add=True)
        pltpu.make_async_copy(x_ref.at[src_all.at[b]], rows1, sem1).wait()

        @pl.when(j < _HC // 2 - 1)
        def _next():
            pltpu.async_copy(x_ref.at[src_all.at[a + 2]], rows0, sem0)

        pltpu.sync_copy(rows1, agg_sh.at[dst_all.at[b]], add=True)
        return carry

    for h in range(_NH):
        if h > 0:
            pltpu.sync_copy(src_ref.at[wid, h], src_all)
            pltpu.sync_copy(dst_ref.at[wid, h], dst_all)
        pltpu.async_copy(x_ref.at[src_all.at[0]], rows0, sem0)
        lax.fori_loop(0, _HC // 2, chunk_body, 0)

    plsc.subcore_barrier()

    pltpu.sync_copy(agg_sh.at[pl.ds(s * _RB, _RB)],
                    out_ref.at[c, pl.ds(s * _RB, _RB)])

    @pl.when(s == _NS - 1)
    def _copy_rem():
        pltpu.sync_copy(agg_sh.at[pl.ds(_RB * _NS, _RREM)],
                        out_ref.at[c, pl.ds(_RB * _NS, _RREM)])


@functools.lru_cache(maxsize=1)
def _sc_agg():
    return pl.kernel(
        _sc_agg_body,
        out_type=jax.ShapeDtypeStruct((_NC, _N, _D), jnp.float32),
        mesh=plsc.VectorSubcoreMesh(core_axis_name="c", subcore_axis_name="s",
                                    num_cores=_NC, num_subcores=_NS),
        scratch_types=[
            pltpu.VMEM((_HC, _CHUNK), jnp.int32),
            pltpu.VMEM((_HC, _CHUNK), jnp.int32),
            pltpu.VMEM((_CHUNK, _D), jnp.float32),
            pltpu.VMEM((_CHUNK, _D), jnp.float32),
            pltpu.VMEM((_ZR, _D), jnp.float32),
            pltpu.VMEM_SHARED((_N, _D), jnp.float32),
            pltpu.SemaphoreType.DMA,
            pltpu.SemaphoreType.DMA,
            pltpu.SemaphoreType.DMA,
        ],
    )


def _mlp_body(x_ref, agg_ref, w1_ref, b1_ref, w2_ref, b2_ref, w3_ref, b3_ref,
              out_ref):
    h = agg_ref[0] + agg_ref[1] + x_ref[...]
    h = jnp.dot(h, w1_ref[...], preferred_element_type=jnp.float32)
    h = jnp.maximum(h + b1_ref[...], 0.0)
    h = jnp.dot(h, w2_ref[...], preferred_element_type=jnp.float32)
    h = jnp.maximum(h + b2_ref[...], 0.0)
    colsum = jnp.sum(h, axis=0, keepdims=True)
    out_ref[...] = (jnp.dot(colsum, w3_ref[...],
                            preferred_element_type=jnp.float32) + b3_ref[...])


_mlp = pl.pallas_call(
    _mlp_body,
    out_shape=jax.ShapeDtypeStruct((1, _D), jnp.float32),
)


def kernel(x, edge_index, W1, b1, W2, b2, W3, b3):
    ei = edge_index.astype(jnp.int32)
    src4 = ei[0].reshape(_NW, _NH, _HC, _CHUNK)
    dst4 = ei[1].reshape(_NW, _NH, _HC, _CHUNK)
    agg = _sc_agg()(src4, dst4, x)
    w3t = jnp.zeros((_D, _D), jnp.float32).at[:, :_NCLS].set(W3.T)
    b3p = jnp.zeros((1, _D), jnp.float32).at[0, :_NCLS].set(b3 * _N)
    y = _mlp(x, agg, W1.T, b1.reshape(1, _D), W2.T, b2.reshape(1, _D),
             w3t, b3p)
    return y[0, :_NCLS]


# speedup vs baseline: 9.6064x; 1.0007x over previous
import functools

import jax
import jax.numpy as jnp
from jax import lax
from jax.experimental import pallas as pl
from jax.experimental.pallas import tpu as pltpu
from jax.experimental.pallas import tpu_sc as plsc

_N = 10000
_D = 128
_E = 320000
_NCLS = 6
_NC = 2
_NS = 16
_NW = _NC * _NS
_EPT = _E // _NW
_CHUNK = 100
_NCHUNK = _EPT // _CHUNK
_NH = 2
_HC = _NCHUNK // _NH
_RB = 624
_RREM = _N - _RB * _NS
_ZR = 16


def _sc_agg_body(src_ref, dst_ref, x_ref, out_ref, src_all, dst_all, rows0,
                 rows1, zero_v, agg_sh, sem_i, sem0, sem1, sem_t0, sem_t1):
    c = lax.axis_index("c")
    s = lax.axis_index("s")
    wid = c * _NS + s

    idx_cp0 = pltpu.async_copy(src_ref.at[wid, 0], src_all, sem_i)
    idx_cp1 = pltpu.async_copy(dst_ref.at[wid, 0], dst_all, sem_i)

    zvec = jnp.zeros((16,), jnp.float32)
    for i in range(_ZR):
        for j in range(_D // 16):
            zero_v[i, pl.ds(j * 16, 16)] = zvec

    def zero_body(i, carry):
        pltpu.sync_copy(zero_v, agg_sh.at[pl.ds(s * _RB + i * _ZR, _ZR)])
        return carry

    lax.fori_loop(0, _RB // _ZR, zero_body, 0)

    @pl.when(s == _NS - 1)
    def _zero_rem():
        pltpu.sync_copy(zero_v, agg_sh.at[pl.ds(_RB * _NS, _RREM)])

    idx_cp0.wait()
    idx_cp1.wait()
    plsc.subcore_barrier()

    def chunk_body(j, carry):
        a = 2 * j
        b = a + 1
        pltpu.make_async_copy(x_ref.at[src_all.at[a]], rows0, sem0).wait()

        @pl.when(j > 0)
        def _rows1_free():
            pltpu.make_async_copy(
                rows1, agg_sh.at[dst_all.at[lax.max(a - 1, 0)]],
                sem_t1).wait()

        pltpu.async_copy(x_ref.at[src_all.at[b]], rows1, sem1)
        pltpu.async_copy(rows0, agg_sh.at[dst_all.at[a]], sem_t0, add=True)
        pltpu.make_async_copy(x_ref.at[src_all.at[b]], rows1, sem1).wait()
        pltpu.make_async_copy(rows0, agg_sh.at[dst_all.at[a]], sem_t0).wait()

        @pl.when(j < _HC // 2 - 1)
        def _next():
            pltpu.async_copy(x_ref.at[src_all.at[a + 2]], rows0, sem0)

        pltpu.async_copy(rows1, agg_sh.at[dst_all.at[b]], sem_t1, add=True)
        return carry

    def wait_last_scatter():
        pltpu.make_async_copy(rows1, agg_sh.at[dst_all.at[_HC - 1]],
                              sem_t1).wait()

    for h in range(_NH):
        if h > 0:
            wait_last_scatter()
            pltpu.sync_copy(src_ref.at[wid, h], src_all)
            pltpu.sync_copy(dst_ref.at[wid, h], dst_all)
        pltpu.async_copy(x_ref.at[src_all.at[0]], rows0, sem0)
        lax.fori_loop(0, _HC // 2, chunk_body, 0)

    wait_last_scatter()
    plsc.subcore_barrier()

    pltpu.sync_copy(agg_sh.at[pl.ds(s * _RB, _RB)],
                    out_ref.at[c, pl.ds(s * _RB, _RB)])

    @pl.when(s == _NS - 1)
    def _copy_rem():
        pltpu.sync_copy(agg_sh.at[pl.ds(_RB * _NS, _RREM)],
                        out_ref.at[c, pl.ds(_RB * _NS, _RREM)])


@functools.lru_cache(maxsize=1)
def _sc_agg():
    return pl.kernel(
        _sc_agg_body,
        out_type=jax.ShapeDtypeStruct((_NC, _N, _D), jnp.float32),
        mesh=plsc.VectorSubcoreMesh(core_axis_name="c", subcore_axis_name="s",
                                    num_cores=_NC, num_subcores=_NS),
        scratch_types=[
            pltpu.VMEM((_HC, _CHUNK), jnp.int32),
            pltpu.VMEM((_HC, _CHUNK), jnp.int32),
            pltpu.VMEM((_CHUNK, _D), jnp.float32),
            pltpu.VMEM((_CHUNK, _D), jnp.float32),
            pltpu.VMEM((_ZR, _D), jnp.float32),
            pltpu.VMEM_SHARED((_N, _D), jnp.float32),
            pltpu.SemaphoreType.DMA,
            pltpu.SemaphoreType.DMA,
            pltpu.SemaphoreType.DMA,
            pltpu.SemaphoreType.DMA,
            pltpu.SemaphoreType.DMA,
        ],
    )


def _mlp_body(x_ref, agg_ref, w1_ref, b1_ref, w2_ref, b2_ref, w3_ref, b3_ref,
              out_ref):
    h = agg_ref[0] + agg_ref[1] + x_ref[...]
    h = jnp.dot(h, w1_ref[...], preferred_element_type=jnp.float32)
    h = jnp.maximum(h + b1_ref[...], 0.0)
    h = jnp.dot(h, w2_ref[...], preferred_element_type=jnp.float32)
    h = jnp.maximum(h + b2_ref[...], 0.0)
    colsum = jnp.sum(h, axis=0, keepdims=True)
    out_ref[...] = (jnp.dot(colsum, w3_ref[...],
                            preferred_element_type=jnp.float32) + b3_ref[...])


_mlp = pl.pallas_call(
    _mlp_body,
    out_shape=jax.ShapeDtypeStruct((1, _D), jnp.float32),
)


def kernel(x, edge_index, W1, b1, W2, b2, W3, b3):
    ei = edge_index.astype(jnp.int32)
    src4 = ei[0].reshape(_NW, _NH, _HC, _CHUNK)
    dst4 = ei[1].reshape(_NW, _NH, _HC, _CHUNK)
    agg = _sc_agg()(src4, dst4, x)
    w3t = jnp.zeros((_D, _D), jnp.float32).at[:, :_NCLS].set(W3.T)
    b3p = jnp.zeros((1, _D), jnp.float32).at[0, :_NCLS].set(b3 * _N)
    y = _mlp(x, agg, W1.T, b1.reshape(1, _D), W2.T, b2.reshape(1, _D),
             w3t, b3p)
    return y[0, :_NCLS]


# speedup vs baseline: 10.2338x; 1.0653x over previous
import functools

import jax
import jax.numpy as jnp
from jax import lax
from jax.experimental import pallas as pl
from jax.experimental.pallas import tpu as pltpu
from jax.experimental.pallas import tpu_sc as plsc

_N = 10000
_D = 128
_E = 320000
_NCLS = 6
_NC = 2
_NS = 16
_NW = _NC * _NS
_EPT = _E // _NW
_CHUNK = 50
_NCHUNK = _EPT // _CHUNK
_NP = 5
_PC = _NCHUNK // _NP
_NB = 4
_RB = 624
_RREM = _N - _RB * _NS
_ZR = 16


def _sc_agg_body(src_ref, dst_ref, x_ref, out_ref, src_all, dst_all,
                 rows0, rows1, rows2, rows3, zero_v, agg_sh, sem_i,
                 sg0, sg1, sg2, sg3, ss0, ss1, ss2, ss3):
    rows = (rows0, rows1, rows2, rows3)
    sg = (sg0, sg1, sg2, sg3)
    ss = (ss0, ss1, ss2, ss3)
    c = lax.axis_index("c")
    s = lax.axis_index("s")
    wid = c * _NS + s

    idx_cp0 = pltpu.async_copy(src_ref.at[wid, 0], src_all, sem_i)
    idx_cp1 = pltpu.async_copy(dst_ref.at[wid, 0], dst_all, sem_i)

    zvec = jnp.zeros((16,), jnp.float32)
    for i in range(_ZR):
        for j in range(_D // 16):
            zero_v[i, pl.ds(j * 16, 16)] = zvec

    def zero_body(i, carry):
        pltpu.sync_copy(zero_v, agg_sh.at[pl.ds(s * _RB + i * _ZR, _ZR)])
        return carry

    lax.fori_loop(0, _RB // _ZR, zero_body, 0)

    @pl.when(s == _NS - 1)
    def _zero_rem():
        pltpu.sync_copy(zero_v, agg_sh.at[pl.ds(_RB * _NS, _RREM)])

    idx_cp0.wait()
    idx_cp1.wait()
    plsc.subcore_barrier()

    def gather(ch, r, sem):
        pltpu.async_copy(x_ref.at[src_all.at[ch]], rows[r], sem)

    def chunk_body(j, carry):
        base = _NB * j
        for r in range(_NB):
            pltpu.make_async_copy(x_ref.at[src_all.at[base + r]],
                                  rows[r], sg[r]).wait()
            pltpu.async_copy(rows[r], agg_sh.at[dst_all.at[base + r]],
                             ss[r], add=True)
        for r in range(_NB):
            pltpu.make_async_copy(rows[r],
                                  agg_sh.at[dst_all.at[base + r]],
                                  ss[r]).wait()

            @pl.when(j < _PC // _NB - 1)
            def _next(r=r, base=base):
                gather(base + _NB + r, r, sg[r])

        return carry

    for p in range(_NP):
        if p > 0:
            pltpu.sync_copy(src_ref.at[wid, p], src_all)
            pltpu.sync_copy(dst_ref.at[wid, p], dst_all)
        for r in range(_NB):
            gather(r, r, sg[r])
        lax.fori_loop(0, _PC // _NB, chunk_body, 0)

    plsc.subcore_barrier()

    pltpu.sync_copy(agg_sh.at[pl.ds(s * _RB, _RB)],
                    out_ref.at[c, pl.ds(s * _RB, _RB)])

    @pl.when(s == _NS - 1)
    def _copy_rem():
        pltpu.sync_copy(agg_sh.at[pl.ds(_RB * _NS, _RREM)],
                        out_ref.at[c, pl.ds(_RB * _NS, _RREM)])


@functools.lru_cache(maxsize=1)
def _sc_agg():
    return pl.kernel(
        _sc_agg_body,
        out_type=jax.ShapeDtypeStruct((_NC, _N, _D), jnp.float32),
        mesh=plsc.VectorSubcoreMesh(core_axis_name="c", subcore_axis_name="s",
                                    num_cores=_NC, num_subcores=_NS),
        scratch_types=[
            pltpu.VMEM((_PC, _CHUNK), jnp.int32),
            pltpu.VMEM((_PC, _CHUNK), jnp.int32),
            pltpu.VMEM((_CHUNK, _D), jnp.float32),
            pltpu.VMEM((_CHUNK, _D), jnp.float32),
            pltpu.VMEM((_CHUNK, _D), jnp.float32),
            pltpu.VMEM((_CHUNK, _D), jnp.float32),
            pltpu.VMEM((_ZR, _D), jnp.float32),
            pltpu.VMEM_SHARED((_N, _D), jnp.float32),
        ] + [pltpu.SemaphoreType.DMA] * 9,
    )


def _mlp_body(x_ref, agg_ref, w1_ref, b1_ref, w2_ref, b2_ref, w3_ref, b3_ref,
              out_ref):
    h = agg_ref[0] + agg_ref[1] + x_ref[...]
    h = jnp.dot(h, w1_ref[...], preferred_element_type=jnp.float32)
    h = jnp.maximum(h + b1_ref[...], 0.0)
    h = jnp.dot(h, w2_ref[...], preferred_element_type=jnp.float32)
    h = jnp.maximum(h + b2_ref[...], 0.0)
    colsum = jnp.sum(h, axis=0, keepdims=True)
    out_ref[...] = (jnp.dot(colsum, w3_ref[...],
                            preferred_element_type=jnp.float32) + b3_ref[...])


_mlp = pl.pallas_call(
    _mlp_body,
    out_shape=jax.ShapeDtypeStruct((1, _D), jnp.float32),
)


def kernel(x, edge_index, W1, b1, W2, b2, W3, b3):
    ei = edge_index.astype(jnp.int32)
    src4 = ei[0].reshape(_NW, _NP, _PC, _CHUNK)
    dst4 = ei[1].reshape(_NW, _NP, _PC, _CHUNK)
    agg = _sc_agg()(src4, dst4, x)
    w3t = jnp.zeros((_D, _D), jnp.float32).at[:, :_NCLS].set(W3.T)
    b3p = jnp.zeros((1, _D), jnp.float32).at[0, :_NCLS].set(b3 * _N)
    y = _mlp(x, agg, W1.T, b1.reshape(1, _D), W2.T, b2.reshape(1, _D),
             w3t, b3p)
    return y[0, :_NCLS]


# speedup vs baseline: 10.2676x; 1.0033x over previous
import functools

import jax
import jax.numpy as jnp
from jax import lax
from jax.experimental import pallas as pl
from jax.experimental.pallas import tpu as pltpu
from jax.experimental.pallas import tpu_sc as plsc

_N = 10000
_D = 128
_E = 320000
_NCLS = 6
_NC = 2
_NS = 16
_NW = _NC * _NS
_EPT = _E // _NW
_CHUNK = 50
_NCHUNK = _EPT // _CHUNK
_NP = 5
_PC = _NCHUNK // _NP
_NB = 4
_RB = 624
_RREM = _N - _RB * _NS
_ZR = 16


def _sc_agg_body(src_ref, dst_ref, x_ref, out_ref, src_all, dst_all,
                 rows0, rows1, rows2, rows3, zero_v, agg_sh, sem_i,
                 sg0, sg1, sg2, sg3, ss0, ss1, ss2, ss3):
    rows = (rows0, rows1, rows2, rows3)
    sg = (sg0, sg1, sg2, sg3)
    ss = (ss0, ss1, ss2, ss3)
    c = lax.axis_index("c")
    s = lax.axis_index("s")
    wid = c * _NS + s

    idx_cp0 = pltpu.async_copy(src_ref.at[wid, 0], src_all, sem_i)
    idx_cp1 = pltpu.async_copy(dst_ref.at[wid, 0], dst_all, sem_i)

    @pl.when(c == 0)
    def _init_x():
        pltpu.sync_copy(x_ref.at[pl.ds(s * _RB, _RB)],
                        agg_sh.at[pl.ds(s * _RB, _RB)])

        @pl.when(s == _NS - 1)
        def _init_x_rem():
            pltpu.sync_copy(x_ref.at[pl.ds(_RB * _NS, _RREM)],
                            agg_sh.at[pl.ds(_RB * _NS, _RREM)])

    @pl.when(c == 1)
    def _init_zero():
        zvec = jnp.zeros((16,), jnp.float32)
        for i in range(_ZR):
            for j in range(_D // 16):
                zero_v[i, pl.ds(j * 16, 16)] = zvec

        def zero_body(i, carry):
            pltpu.sync_copy(zero_v, agg_sh.at[pl.ds(s * _RB + i * _ZR, _ZR)])
            return carry

        lax.fori_loop(0, _RB // _ZR, zero_body, 0)

        @pl.when(s == _NS - 1)
        def _zero_rem():
            pltpu.sync_copy(zero_v, agg_sh.at[pl.ds(_RB * _NS, _RREM)])

    idx_cp0.wait()
    idx_cp1.wait()
    plsc.subcore_barrier()

    def gather(ch, r, sem):
        pltpu.async_copy(x_ref.at[src_all.at[ch]], rows[r], sem)

    def chunk_body(j, carry):
        base = _NB * j
        for r in range(_NB):
            pltpu.make_async_copy(x_ref.at[src_all.at[base + r]],
                                  rows[r], sg[r]).wait()
            pltpu.async_copy(rows[r], agg_sh.at[dst_all.at[base + r]],
                             ss[r], add=True)
        for r in range(_NB):
            pltpu.make_async_copy(rows[r],
                                  agg_sh.at[dst_all.at[base + r]],
                                  ss[r]).wait()

            @pl.when(j < _PC // _NB - 1)
            def _next(r=r, base=base):
                gather(base + _NB + r, r, sg[r])

        return carry

    for p in range(_NP):
        if p > 0:
            pltpu.sync_copy(src_ref.at[wid, p], src_all)
            pltpu.sync_copy(dst_ref.at[wid, p], dst_all)
        for r in range(_NB):
            gather(r, r, sg[r])
        lax.fori_loop(0, _PC // _NB, chunk_body, 0)

    plsc.subcore_barrier()

    pltpu.sync_copy(agg_sh.at[pl.ds(s * _RB, _RB)],
                    out_ref.at[c, pl.ds(s * _RB, _RB)])

    @pl.when(s == _NS - 1)
    def _copy_rem():
        pltpu.sync_copy(agg_sh.at[pl.ds(_RB * _NS, _RREM)],
                        out_ref.at[c, pl.ds(_RB * _NS, _RREM)])


@functools.lru_cache(maxsize=1)
def _sc_agg():
    return pl.kernel(
        _sc_agg_body,
        out_type=jax.ShapeDtypeStruct((_NC, _N, _D), jnp.float32),
        mesh=plsc.VectorSubcoreMesh(core_axis_name="c", subcore_axis_name="s",
                                    num_cores=_NC, num_subcores=_NS),
        scratch_types=[
            pltpu.VMEM((_PC, _CHUNK), jnp.int32),
            pltpu.VMEM((_PC, _CHUNK), jnp.int32),
            pltpu.VMEM((_CHUNK, _D), jnp.float32),
            pltpu.VMEM((_CHUNK, _D), jnp.float32),
            pltpu.VMEM((_CHUNK, _D), jnp.float32),
            pltpu.VMEM((_CHUNK, _D), jnp.float32),
            pltpu.VMEM((_ZR, _D), jnp.float32),
            pltpu.VMEM_SHARED((_N, _D), jnp.float32),
        ] + [pltpu.SemaphoreType.DMA] * 9,
    )


def _mlp_body(agg_ref, w1_ref, b1_ref, w2_ref, b2_ref, w3_ref, b3_ref,
              out_ref):
    dn = (((1,), (1,)), ((), ()))
    h = agg_ref[0] + agg_ref[1]
    h = lax.dot_general(h, w1_ref[...], dn, preferred_element_type=jnp.float32)
    h = jnp.maximum(h + b1_ref[...], 0.0)
    h = lax.dot_general(h, w2_ref[...], dn, preferred_element_type=jnp.float32)
    h = jnp.maximum(h + b2_ref[...], 0.0)
    colsum = jnp.sum(h, axis=0, keepdims=True)
    out_ref[...] = (lax.dot_general(colsum, w3_ref[...], dn,
                                    preferred_element_type=jnp.float32)
                    + b3_ref[...] * float(_N))


_mlp = pl.pallas_call(
    _mlp_body,
    out_shape=jax.ShapeDtypeStruct((1, _NCLS), jnp.float32),
)


def kernel(x, edge_index, W1, b1, W2, b2, W3, b3):
    ei = edge_index.astype(jnp.int32)
    src4 = ei[0].reshape(_NW, _NP, _PC, _CHUNK)
    dst4 = ei[1].reshape(_NW, _NP, _PC, _CHUNK)
    agg = _sc_agg()(src4, dst4, x)
    y = _mlp(agg, W1, b1.reshape(1, _D), W2, b2.reshape(1, _D),
             W3, b3.reshape(1, _NCLS))
    return y[0]
